# baseline (speedup 1.0000x reference)
import jax
import jax.numpy as jnp
from jax import lax
from jax.experimental import pallas as pl
from jax.experimental.pallas import tpu as pltpu
from jax.experimental.pallas import tpu_sc as plsc

B = 16384
F = 128
NC = 2
NS = 16
NW = NC * NS
BPW = B // NW
CHUNK = 128
NCHUNK = BPW // CHUNK
GROUPS = CHUNK // 16


def _sc_body(users, items, uw, iw, ub, ib, out,
             uidx, iidx, urows0, urows1, irows0, irows1,
             ubv0, ubv1, ibv0, ibv1, outv, sems):
    wid = lax.axis_index("s") * NC + lax.axis_index("c")
    base = wid * BPW

    urows = (urows0, urows1)
    irows = (irows0, irows1)
    ubv = (ubv0, ubv1)
    ibv = (ibv0, ibv1)

    hu = pltpu.async_copy(users.at[pl.ds(base, BPW)], uidx, sems.at[0])
    hi = pltpu.async_copy(items.at[pl.ds(base, BPW)], iidx, sems.at[1])
    hu.wait()
    hi.wait()

    def issue(c, slot):
        uc = uidx.at[pl.ds(c * CHUNK, CHUNK)]
        ic = iidx.at[pl.ds(c * CHUNK, CHUNK)]
        return [
            pltpu.async_copy(uw.at[uc], urows[slot], sems.at[slot]),
            pltpu.async_copy(iw.at[ic], irows[slot], sems.at[slot]),
            pltpu.async_copy(ub.at[uc], ubv[slot], sems.at[slot]),
            pltpu.async_copy(ib.at[ic], ibv[slot], sems.at[slot]),
        ]

    def compute(c, slot, g_lo=0, g_hi=GROUPS):
        ur = urows[slot]
        ir = irows[slot]
        ubc = ubv[slot]
        ibc = ibv[slot]

        lane = lax.iota(jnp.int32, 16)

        def gbody(g, carry):
            def tbody(t, sums):
                for j in range(8):
                    r = g * 16 + t * 8 + j
                    acc = ur[r, pl.ds(0, 16)] * ir[r, pl.ds(0, 16)]
                    for k in range(1, F // 16):
                        acc = acc + (ur[r, pl.ds(k * 16, 16)]
                                     * ir[r, pl.ds(k * 16, 16)])
                    s = jnp.sum(acc)
                    sums = jnp.where(lane == t * 8 + j,
                                     jnp.full((16,), s), sums)
                return sums

            sums = lax.fori_loop(0, 2, tbody, jnp.zeros((16,), jnp.float32))
            res = sums + ubc[pl.ds(g * 16, 16)] + ibc[pl.ds(g * 16, 16)]
            outv[pl.ds(c * CHUNK + g * 16, 16)] = res
            return carry

        lax.fori_loop(g_lo, g_hi, gbody, 0)

    SUB = 4
    RS = CHUNK // SUB
    h0 = [pltpu.async_copy(ub.at[uidx.at[pl.ds(0, CHUNK)]], ubv[0],
                           sems.at[0]),
          pltpu.async_copy(ib.at[iidx.at[pl.ds(0, CHUNK)]], ibv[0],
                           sems.at[0])]
    subs = []
    for p in range(SUB):
        subs.append((
            pltpu.async_copy(uw.at[uidx.at[pl.ds(p * RS, RS)]],
                             urows[0].at[pl.ds(p * RS, RS)], sems.at[0]),
            pltpu.async_copy(iw.at[iidx.at[pl.ds(p * RS, RS)]],
                             irows[0].at[pl.ds(p * RS, RS)], sems.at[0]),
        ))
    handles = issue(1, 1)
    for h in h0:
        h.wait()
    gpp = GROUPS // SUB
    for p in range(SUB):
        subs[p][0].wait()
        subs[p][1].wait()
        compute(0, 0, p * gpp, (p + 1) * gpp)

    for c in range(1, NCHUNK):
        slot = c % 2
        for h in handles:
            h.wait()
        if c + 1 < NCHUNK:
            handles = issue(c + 1, 1 - slot)
        compute(c, slot)

    def sbody(q, carry):
        for j in range(4):
            res = outv[pl.ds(q * 64 + j * 16, 16)]
            outv[pl.ds(q * 64 + j * 16, 16)] = 5.0 / (1.0 + jnp.exp(-res))
        return carry

    lax.fori_loop(0, BPW // 64, sbody, 0)

    pltpu.sync_copy(outv, out.at[pl.ds(base, BPW)])


@jax.jit
def kernel(users, items, u_weight, i_weight, u_bias, i_bias):
    mesh = plsc.VectorSubcoreMesh(core_axis_name="c", subcore_axis_name="s")
    run = pl.kernel(
        _sc_body,
        out_type=jax.ShapeDtypeStruct((B,), jnp.float32),
        mesh=mesh,
        compiler_params=pltpu.CompilerParams(
            needs_layout_passes=False,
            disable_bounds_checks=True,
            disable_semaphore_checks=True,
        ),
        scratch_types=[
            pltpu.VMEM((BPW,), jnp.int32),
            pltpu.VMEM((BPW,), jnp.int32),
            pltpu.VMEM((CHUNK, F), jnp.float32),
            pltpu.VMEM((CHUNK, F), jnp.float32),
            pltpu.VMEM((CHUNK, F), jnp.float32),
            pltpu.VMEM((CHUNK, F), jnp.float32),
            pltpu.VMEM((CHUNK,), jnp.float32),
            pltpu.VMEM((CHUNK,), jnp.float32),
            pltpu.VMEM((CHUNK,), jnp.float32),
            pltpu.VMEM((CHUNK,), jnp.float32),
            pltpu.VMEM((BPW,), jnp.float32),
            pltpu.SemaphoreType.DMA((2,)),
        ],
    )
    return run(users.astype(jnp.int32), items.astype(jnp.int32),
               u_weight, i_weight, u_bias.reshape(-1), i_bias.reshape(-1))


# speedup vs baseline: 1.0018x; 1.0018x over previous
import jax
import jax.numpy as jnp
from jax import lax
from jax.experimental import pallas as pl
from jax.experimental.pallas import tpu as pltpu
from jax.experimental.pallas import tpu_sc as plsc

B = 16384
F = 128
NC = 2
NS = 16
NW = NC * NS
BPW = B // NW
CHUNK = 128
NCHUNK = BPW // CHUNK
GROUPS = CHUNK // 16


def _sc_body(users, items, uw, iw, ub, ib, out,
             uidx, iidx, urows0, urows1, irows0, irows1,
             ubv, ibv, outv, sems):
    wid = lax.axis_index("s") * NC + lax.axis_index("c")
    base = wid * BPW

    urows = (urows0, urows1)
    irows = (irows0, irows1)

    hu = pltpu.async_copy(users.at[pl.ds(base, BPW)], uidx, sems.at[0])
    hi = pltpu.async_copy(items.at[pl.ds(base, BPW)], iidx, sems.at[1])
    hu.wait()
    hi.wait()

    hb = [pltpu.async_copy(ub.at[uidx], ubv, sems.at[0]),
          pltpu.async_copy(ib.at[iidx], ibv, sems.at[0])]

    def issue(c, slot):
        uc = uidx.at[pl.ds(c * CHUNK, CHUNK)]
        ic = iidx.at[pl.ds(c * CHUNK, CHUNK)]
        return [
            pltpu.async_copy(uw.at[uc], urows[slot], sems.at[slot]),
            pltpu.async_copy(iw.at[ic], irows[slot], sems.at[slot]),
        ]

    def compute(c, slot, g_lo=0, g_hi=GROUPS):
        ur = urows[slot]
        ir = irows[slot]

        lane = lax.iota(jnp.int32, 16)

        def gbody(g, carry):
            def tbody(t, sums):
                for j in range(8):
                    r = g * 16 + t * 8 + j
                    acc = ur[r, pl.ds(0, 16)] * ir[r, pl.ds(0, 16)]
                    for k in range(1, F // 16):
                        acc = acc + (ur[r, pl.ds(k * 16, 16)]
                                     * ir[r, pl.ds(k * 16, 16)])
                    s = jnp.sum(acc)
                    sums = jnp.where(lane == t * 8 + j,
                                     jnp.full((16,), s), sums)
                return sums

            sums = lax.fori_loop(0, 2, tbody, jnp.zeros((16,), jnp.float32))
            off = c * CHUNK + g * 16
            res = sums + ubv[pl.ds(off, 16)] + ibv[pl.ds(off, 16)]
            y = 5.0 / (1.0 + jnp.exp(-res))
            outv[pl.ds(off, 16)] = y
            return carry

        lax.fori_loop(g_lo, g_hi, gbody, 0)

    handles = issue(0, 0)
    for c in range(NCHUNK):
        slot = c % 2
        for h in handles:
            h.wait()
        if c == 0:
            for h in hb:
                h.wait()
        if c + 1 < NCHUNK:
            handles = issue(c + 1, 1 - slot)
        compute(c, slot)

    pltpu.sync_copy(outv, out.at[pl.ds(base, BPW)])


@jax.jit
def kernel(users, items, u_weight, i_weight, u_bias, i_bias):
    mesh = plsc.VectorSubcoreMesh(core_axis_name="c", subcore_axis_name="s",
                                  num_cores=NC, num_subcores=NS)
    run = pl.kernel(
        _sc_body,
        out_type=jax.ShapeDtypeStruct((B,), jnp.float32),
        mesh=mesh,
        compiler_params=pltpu.CompilerParams(
            needs_layout_passes=False,
            disable_bounds_checks=True,
            disable_semaphore_checks=True,
        ),
        scratch_types=[
            pltpu.VMEM((BPW,), jnp.int32),
            pltpu.VMEM((BPW,), jnp.int32),
            pltpu.VMEM((CHUNK, F), jnp.float32),
            pltpu.VMEM((CHUNK, F), jnp.float32),
            pltpu.VMEM((CHUNK, F), jnp.float32),
            pltpu.VMEM((CHUNK, F), jnp.float32),
            pltpu.VMEM((BPW,), jnp.float32),
            pltpu.VMEM((BPW,), jnp.float32),
            pltpu.VMEM((BPW,), jnp.float32),
            pltpu.SemaphoreType.DMA((2,)),
        ],
    )
    return run(users.astype(jnp.int32), items.astype(jnp.int32),
               u_weight, i_weight, u_bias.reshape(-1), i_bias.reshape(-1))


# speedup vs baseline: 1.0108x; 1.0090x over previous
import jax
import jax.numpy as jnp
from jax import lax
from jax.experimental import pallas as pl
from jax.experimental.pallas import tpu as pltpu
from jax.experimental.pallas import tpu_sc as plsc

B = 16384
F = 128
NC = 2
NS = 16
NW = NC * NS
BPW = B // NW
CHUNK = 128
NCHUNK = BPW // CHUNK
GROUPS = CHUNK // 16


def _sc_body(users, items, uw, iw, ub, ib, out,
             uidx, iidx, urows0, urows1, irows0, irows1,
             ubv, ibv, outv, sems):
    wid = lax.axis_index("s") * NC + lax.axis_index("c")
    base = wid * BPW

    urows = (urows0, urows1)
    irows = (irows0, irows1)

    hu = pltpu.async_copy(users.at[pl.ds(base, BPW)], uidx, sems.at[0])
    hi = pltpu.async_copy(items.at[pl.ds(base, BPW)], iidx, sems.at[1])
    hu.wait()
    hi.wait()

    def issue(c, slot):
        uc = uidx.at[pl.ds(c * CHUNK, CHUNK)]
        ic = iidx.at[pl.ds(c * CHUNK, CHUNK)]
        return [
            pltpu.async_copy(uw.at[uc], urows[slot], sems.at[slot]),
            pltpu.async_copy(iw.at[ic], irows[slot], sems.at[slot]),
        ]

    def compute(c, slot, g_lo=0, g_hi=GROUPS):
        ur = urows[slot]
        ir = irows[slot]

        lane = lax.iota(jnp.int32, 16)

        def gbody(g, carry):
            def tbody(t, sums):
                for j in range(8):
                    r = g * 16 + t * 8 + j
                    acc = ur[r, pl.ds(0, 16)] * ir[r, pl.ds(0, 16)]
                    for k in range(1, F // 16):
                        acc = acc + (ur[r, pl.ds(k * 16, 16)]
                                     * ir[r, pl.ds(k * 16, 16)])
                    s = jnp.sum(acc)
                    sums = jnp.where(lane == t * 8 + j,
                                     jnp.full((16,), s), sums)
                return sums

            sums = lax.fori_loop(0, 2, tbody, jnp.zeros((16,), jnp.float32))
            off = c * CHUNK + g * 16
            res = sums + ubv[pl.ds(off, 16)] + ibv[pl.ds(off, 16)]
            y = 5.0 / (1.0 + jnp.exp(-res))
            outv[pl.ds(off, 16)] = y
            return carry

        lax.fori_loop(g_lo, g_hi, gbody, 0)

    HALF = CHUNK // 2
    w0 = [pltpu.async_copy(uw.at[uidx.at[pl.ds(0, HALF)]],
                           urows[0].at[pl.ds(0, HALF)], sems.at[0]),
          pltpu.async_copy(iw.at[iidx.at[pl.ds(0, HALF)]],
                           irows[0].at[pl.ds(0, HALF)], sems.at[0])]
    w1 = [pltpu.async_copy(uw.at[uidx.at[pl.ds(HALF, HALF)]],
                           urows[0].at[pl.ds(HALF, HALF)], sems.at[0]),
          pltpu.async_copy(iw.at[iidx.at[pl.ds(HALF, HALF)]],
                           irows[0].at[pl.ds(HALF, HALF)], sems.at[0])]
    hb = [pltpu.async_copy(ub.at[uidx], ubv, sems.at[1]),
          pltpu.async_copy(ib.at[iidx], ibv, sems.at[1])]
    handles = issue(1, 1)
    for h in w0 + hb:
        h.wait()
    compute(0, 0, 0, GROUPS // 2)
    for h in w1:
        h.wait()
    compute(0, 0, GROUPS // 2, GROUPS)

    for c in range(1, NCHUNK):
        slot = c % 2
        for h in handles:
            h.wait()
        if c + 1 < NCHUNK:
            handles = issue(c + 1, 1 - slot)
        compute(c, slot)

    pltpu.sync_copy(outv, out.at[pl.ds(base, BPW)])


@jax.jit
def kernel(users, items, u_weight, i_weight, u_bias, i_bias):
    mesh = plsc.VectorSubcoreMesh(core_axis_name="c", subcore_axis_name="s",
                                  num_cores=NC, num_subcores=NS)
    run = pl.kernel(
        _sc_body,
        out_type=jax.ShapeDtypeStruct((B,), jnp.float32),
        mesh=mesh,
        compiler_params=pltpu.CompilerParams(
            needs_layout_passes=False,
            disable_bounds_checks=True,
            disable_semaphore_checks=True,
        ),
        scratch_types=[
            pltpu.VMEM((BPW,), jnp.int32),
            pltpu.VMEM((BPW,), jnp.int32),
            pltpu.VMEM((CHUNK, F), jnp.float32),
            pltpu.VMEM((CHUNK, F), jnp.float32),
            pltpu.VMEM((CHUNK, F), jnp.float32),
            pltpu.VMEM((CHUNK, F), jnp.float32),
            pltpu.VMEM((BPW,), jnp.float32),
            pltpu.VMEM((BPW,), jnp.float32),
            pltpu.VMEM((BPW,), jnp.float32),
            pltpu.SemaphoreType.DMA((2,)),
        ],
    )
    return run(users.astype(jnp.int32), items.astype(jnp.int32),
               u_weight, i_weight, u_bias.reshape(-1), i_bias.reshape(-1))


# speedup vs baseline: 1.0125x; 1.0017x over previous
import jax
import jax.numpy as jnp
from jax import lax
from jax.experimental import pallas as pl
from jax.experimental.pallas import tpu as pltpu
from jax.experimental.pallas import tpu_sc as plsc

B = 16384
F = 128
NC = 2
NS = 16
NW = NC * NS
BPW = B // NW
CHUNK = 128
NCHUNK = BPW // CHUNK
GROUPS = CHUNK // 16


def _sc_body(users, items, uw, iw, ub, ib, out,
             uidx, iidx, urows0, urows1, irows0, irows1,
             ubv, ibv, outv, sems):
    wid = lax.axis_index("s") * NC + lax.axis_index("c")
    base = wid * BPW

    urows = (urows0, urows1)
    irows = (irows0, irows1)

    hu = pltpu.async_copy(users.at[pl.ds(base, BPW)], uidx, sems.at[0])
    hi = pltpu.async_copy(items.at[pl.ds(base, BPW)], iidx, sems.at[1])
    hu.wait()
    hi.wait()

    def issue(c, slot):
        uc = uidx.at[pl.ds(c * CHUNK, CHUNK)]
        ic = iidx.at[pl.ds(c * CHUNK, CHUNK)]
        return [
            pltpu.async_copy(uw.at[uc], urows[slot], sems.at[slot]),
            pltpu.async_copy(iw.at[ic], irows[slot], sems.at[slot]),
        ]

    def compute(c, slot, g_lo=0, g_hi=GROUPS):
        ur = urows[slot]
        ir = irows[slot]

        lane = lax.iota(jnp.int32, 16)

        def gbody(g, carry):
            def tbody(t, sums):
                for j in range(8):
                    r = g * 16 + t * 8 + j
                    acc = ur[r, pl.ds(0, 16)] * ir[r, pl.ds(0, 16)]
                    for k in range(1, F // 16):
                        acc = acc + (ur[r, pl.ds(k * 16, 16)]
                                     * ir[r, pl.ds(k * 16, 16)])
                    s = jnp.sum(acc)
                    sums = jnp.where(lane == t * 8 + j,
                                     jnp.full((16,), s), sums)
                return sums

            sums = lax.fori_loop(0, 2, tbody, jnp.zeros((16,), jnp.float32))
            off = c * CHUNK + g * 16
            res = sums + ubv[pl.ds(off, 16)] + ibv[pl.ds(off, 16)]
            y = 5.0 / (1.0 + jnp.exp(-res))
            outv[pl.ds(off, 16)] = y
            return carry

        lax.fori_loop(g_lo, g_hi, gbody, 0)

    HALF = CHUNK // 2
    w0 = [pltpu.async_copy(uw.at[uidx.at[pl.ds(0, HALF)]],
                           urows[0].at[pl.ds(0, HALF)], sems.at[0]),
          pltpu.async_copy(iw.at[iidx.at[pl.ds(0, HALF)]],
                           irows[0].at[pl.ds(0, HALF)], sems.at[0])]
    w1 = [pltpu.async_copy(uw.at[uidx.at[pl.ds(HALF, HALF)]],
                           urows[0].at[pl.ds(HALF, HALF)], sems.at[0]),
          pltpu.async_copy(iw.at[iidx.at[pl.ds(HALF, HALF)]],
                           irows[0].at[pl.ds(HALF, HALF)], sems.at[0])]
    hb = [pltpu.async_copy(ub.at[uidx], ubv, sems.at[1]),
          pltpu.async_copy(ib.at[iidx], ibv, sems.at[1])]
    handles = issue(1, 1)
    for h in w0 + hb:
        h.wait()
    compute(0, 0, 0, GROUPS // 2)
    for h in w1:
        h.wait()
    compute(0, 0, GROUPS // 2, GROUPS)

    def flush(c):
        return pltpu.async_copy(
            outv.at[pl.ds(c * CHUNK, CHUNK)],
            out.at[pl.ds(base + c * CHUNK, CHUNK)], sems.at[2])

    hout = [flush(0)]
    for c in range(1, NCHUNK):
        slot = c % 2
        for h in handles:
            h.wait()
        if c + 1 < NCHUNK:
            handles = issue(c + 1, 1 - slot)
        compute(c, slot)
        hout.append(flush(c))

    for h in hout:
        h.wait()


@jax.jit
def kernel(users, items, u_weight, i_weight, u_bias, i_bias):
    mesh = plsc.VectorSubcoreMesh(core_axis_name="c", subcore_axis_name="s",
                                  num_cores=NC, num_subcores=NS)
    run = pl.kernel(
        _sc_body,
        out_type=jax.ShapeDtypeStruct((B,), jnp.float32),
        mesh=mesh,
        compiler_params=pltpu.CompilerParams(
            needs_layout_passes=False,
            disable_bounds_checks=True,
            disable_semaphore_checks=True,
        ),
        scratch_types=[
            pltpu.VMEM((BPW,), jnp.int32),
            pltpu.VMEM((BPW,), jnp.int32),
            pltpu.VMEM((CHUNK, F), jnp.float32),
            pltpu.VMEM((CHUNK, F), jnp.float32),
            pltpu.VMEM((CHUNK, F), jnp.float32),
            pltpu.VMEM((CHUNK, F), jnp.float32),
            pltpu.VMEM((BPW,), jnp.float32),
            pltpu.VMEM((BPW,), jnp.float32),
            pltpu.VMEM((BPW,), jnp.float32),
            pltpu.SemaphoreType.DMA((3,)),
        ],
    )
    return run(users.astype(jnp.int32), items.astype(jnp.int32),
               u_weight, i_weight, u_bias.reshape(-1), i_bias.reshape(-1))
